# baseline (speedup 1.0000x reference)
import math

import jax
import jax.numpy as jnp
from jax.experimental import pallas as pl

_N = 4096
_D = 64

_ROWS_PROJ = 256
_ROWS_ENC = 256
_ROWS_DEC = 512
_ROWS_ATT = 512


def _proj_kernel(f_ref, w_ref, o_ref):
    o_ref[...] = jnp.dot(f_ref[...], w_ref[...], preferred_element_type=jnp.float32)


def _project(feat, w1):
    n, din = feat.shape
    d = w1.shape[1]
    rows = _ROWS_PROJ
    return pl.pallas_call(
        _proj_kernel,
        grid=(n // rows,),
        in_specs=[
            pl.BlockSpec((rows, din), lambda i: (i, 0)),
            pl.BlockSpec((din, d), lambda i: (0, 0)),
        ],
        out_specs=pl.BlockSpec((rows, d), lambda i: (i, 0)),
        out_shape=jax.ShapeDtypeStruct((n, d), jnp.float32),
    )(feat, w1)


def _enc_kernel(alpha_ref, sp_ref, ft_ref, x_ref, w2_ref, adjb_ref, g_ref):
    s = jax.nn.sigmoid(alpha_ref[0, 0])
    c = s * sp_ref[...] + (1.0 - s) * ft_ref[...]
    adjb_ref[...] = c.astype(jnp.bfloat16)
    h = jnp.maximum(jnp.dot(c, x_ref[...], preferred_element_type=jnp.float32), 0.0)
    g_ref[...] = jnp.dot(h, w2_ref[...], preferred_element_type=jnp.float32)


def _encode(alpha, sp, ft, x, w2):
    rows = _ROWS_ENC
    return pl.pallas_call(
        _enc_kernel,
        grid=(_N // rows,),
        in_specs=[
            pl.BlockSpec((1, 1), lambda i: (0, 0)),
            pl.BlockSpec((rows, _N), lambda i: (i, 0)),
            pl.BlockSpec((rows, _N), lambda i: (i, 0)),
            pl.BlockSpec((_N, _D), lambda i: (0, 0)),
            pl.BlockSpec((_D, _D), lambda i: (0, 0)),
        ],
        out_specs=[
            pl.BlockSpec((rows, _N), lambda i: (i, 0)),
            pl.BlockSpec((rows, _D), lambda i: (i, 0)),
        ],
        out_shape=[
            jax.ShapeDtypeStruct((_N, _N), jnp.bfloat16),
            jax.ShapeDtypeStruct((_N, _D), jnp.float32),
        ],
    )(jnp.reshape(alpha, (1, 1)), sp, ft, x, w2)


def _layernorm(x, g, b):
    mu = jnp.mean(x, axis=-1, keepdims=True)
    v = jnp.mean((x - mu) ** 2, axis=-1, keepdims=True)
    return (x - mu) * jax.lax.rsqrt(v + 1e-5) * g + b


def _dec_kernel(adjb_ref, y_ref, g_ref, b_ref, wq_ref, bq_ref, wk_ref, bk_ref,
                wv_ref, bv_ref, emb_ref, q_ref, k_ref, v_ref):
    acc = jnp.dot(adjb_ref[...], y_ref[...], preferred_element_type=jnp.float32)
    e = _layernorm(acc, g_ref[...], b_ref[...])
    emb_ref[...] = e
    q_ref[...] = jnp.dot(e, wq_ref[...], preferred_element_type=jnp.float32) + bq_ref[...]
    k_ref[...] = jnp.dot(e, wk_ref[...], preferred_element_type=jnp.float32) + bk_ref[...]
    v_ref[...] = jnp.dot(e, wv_ref[...], preferred_element_type=jnp.float32) + bv_ref[...]


def _decode(adjb, y, g, b, wq, bq, wk, bk, wv, bv):
    rows = _ROWS_DEC
    vec = lambda a: jnp.reshape(a, (1, _D))
    mat_spec = pl.BlockSpec((_D, _D), lambda i: (0, 0))
    vec_spec = pl.BlockSpec((1, _D), lambda i: (0, 0))
    out_spec = pl.BlockSpec((rows, _D), lambda i: (i, 0))
    out_shape = jax.ShapeDtypeStruct((_N, _D), jnp.float32)
    return pl.pallas_call(
        _dec_kernel,
        grid=(_N // rows,),
        in_specs=[
            pl.BlockSpec((rows, _N), lambda i: (i, 0)),
            pl.BlockSpec((_N, _D), lambda i: (0, 0)),
            vec_spec, vec_spec,
            mat_spec, vec_spec, mat_spec, vec_spec, mat_spec, vec_spec,
        ],
        out_specs=[out_spec, out_spec, out_spec, out_spec],
        out_shape=[out_shape, out_shape, out_shape, out_shape],
    )(adjb, y.astype(jnp.bfloat16), vec(g), vec(b),
      wq, vec(bq), wk, vec(bk), wv, vec(bv))


def _softmax(s):
    m = jnp.max(s, axis=-1, keepdims=True)
    p = jnp.exp(s - m)
    return p / jnp.sum(p, axis=-1, keepdims=True)


def _fuse_kernel(e1_ref, e2_ref, q12_ref, q21_ref, k12_ref, v12_ref, k21_ref,
                 v21_ref, wo12_ref, bo12_ref, wo21_ref, bo21_ref,
                 n1g_ref, n1b_ref, n2g_ref, n2b_ref,
                 wg_ref, bg_ref, wa_ref, ba_ref, wb_ref, bb_ref,
                 lg_ref, lb_ref, o_ref):
    scale = 1.0 / math.sqrt(float(_D))
    dn = (((1,), (1,)), ((), ()))

    s1 = jax.lax.dot_general(q12_ref[...], k12_ref[...], dn,
                             preferred_element_type=jnp.float32) * scale
    c1 = jnp.dot(jnp.dot(_softmax(s1), v12_ref[...],
                         preferred_element_type=jnp.float32),
                 wo12_ref[...], preferred_element_type=jnp.float32) + bo12_ref[...]
    s2 = jax.lax.dot_general(q21_ref[...], k21_ref[...], dn,
                             preferred_element_type=jnp.float32) * scale
    c2 = jnp.dot(jnp.dot(_softmax(s2), v21_ref[...],
                         preferred_element_type=jnp.float32),
                 wo21_ref[...], preferred_element_type=jnp.float32) + bo21_ref[...]

    e1 = e1_ref[...]
    e2 = e2_ref[...]
    e1e = _layernorm(e1 + 0.2 * c1, n1g_ref[...], n1b_ref[...])
    e2e = _layernorm(e2 + 0.2 * c2, n2g_ref[...], n2b_ref[...])
    cat = jnp.concatenate([e1, e2, e1e, e2e], axis=-1)
    gate = jax.nn.sigmoid(
        jnp.dot(cat, wg_ref[...], preferred_element_type=jnp.float32) + bg_ref[...])
    ha = jnp.dot(cat, wa_ref[...], preferred_element_type=jnp.float32) + ba_ref[...]
    h = ha * 0.5 * (1.0 + jax.lax.erf(ha * (1.0 / math.sqrt(2.0))))
    h = jnp.dot(h, wb_ref[...], preferred_element_type=jnp.float32) + bb_ref[...]
    h = _layernorm(h, lg_ref[...], lb_ref[...])
    o_ref[...] = gate * h + (1.0 - gate) * (e1 + e2) * 0.5


def _fuse(e1, e2, q12, q21, k12, v12, k21, v21, p):
    rows = _ROWS_ATT
    vec = lambda a: jnp.reshape(a, (1, -1))
    blk = pl.BlockSpec((rows, _D), lambda i: (i, 0))
    full = pl.BlockSpec((_N, _D), lambda i: (0, 0))
    mat = lambda a, b: pl.BlockSpec((a, b), lambda i: (0, 0))
    vspec = lambda w: pl.BlockSpec((1, w), lambda i: (0, 0))
    return pl.pallas_call(
        _fuse_kernel,
        grid=(_N // rows,),
        in_specs=[
            blk, blk, blk, blk, full, full, full, full,
            mat(_D, _D), vspec(_D), mat(_D, _D), vspec(_D),
            vspec(_D), vspec(_D), vspec(_D), vspec(_D),
            mat(4 * _D, _D), vspec(_D), mat(4 * _D, 2 * _D), vspec(2 * _D),
            mat(2 * _D, _D), vspec(_D),
            vspec(_D), vspec(_D),
        ],
        out_specs=blk,
        out_shape=jax.ShapeDtypeStruct((_N, _D), jnp.float32),
    )(e1, e2, q12, q21, k12, v12, k21, v21,
      p['a12_Wo'], vec(p['a12_bo']), p['a21_Wo'], vec(p['a21_bo']),
      vec(p['n1_g']), vec(p['n1_b']), vec(p['n2_g']), vec(p['n2_b']),
      p['Wg'], vec(p['bg']), p['Wa'], vec(p['ba']), p['Wb'], vec(p['bb']),
      vec(p['lnf_g']), vec(p['lnf_b']))


def kernel(features_omics1, features_omics2, adj_spatial_omics1, adj_feature_omics1,
           adj_spatial_omics2, adj_feature_omics2, params):
    p = params
    x1 = _project(features_omics1, p['e1_W1'])
    x2 = _project(features_omics2, p['e2_W1'])
    adjb1, g1 = _encode(p['e1_alpha'], adj_spatial_omics1, adj_feature_omics1,
                        x1, p['e1_W2'])
    adjb2, g2 = _encode(p['e2_alpha'], adj_spatial_omics2, adj_feature_omics2,
                        x2, p['e2_W2'])
    emb1, q12, k21, v21 = _decode(adjb1, g1, p['e1_ln_g'], p['e1_ln_b'],
                                  p['a12_Wq'], p['a12_bq'],
                                  p['a21_Wk'], p['a21_bk'],
                                  p['a21_Wv'], p['a21_bv'])
    emb2, q21, k12, v12 = _decode(adjb2, g2, p['e2_ln_g'], p['e2_ln_b'],
                                  p['a21_Wq'], p['a21_bq'],
                                  p['a12_Wk'], p['a12_bk'],
                                  p['a12_Wv'], p['a12_bv'])
    return _fuse(emb1, emb2, q12, q21, k12, v12, k21, v21, p)


# speedup vs baseline: 1.0199x; 1.0199x over previous
import math

import jax
import jax.numpy as jnp
from jax.experimental import pallas as pl

_N = 4096
_D = 64

_ROWS_PROJ = 256
_ROWS_ENC = 256
_ROWS_DEC = 512
_ROWS_ATT = 512


def _proj_kernel(f_ref, w_ref, o_ref):
    o_ref[...] = jnp.dot(f_ref[...], w_ref[...], preferred_element_type=jnp.float32)


def _project(feat, w1):
    n, din = feat.shape
    d = w1.shape[1]
    rows = _ROWS_PROJ
    return pl.pallas_call(
        _proj_kernel,
        grid=(n // rows,),
        in_specs=[
            pl.BlockSpec((rows, din), lambda i: (i, 0)),
            pl.BlockSpec((din, d), lambda i: (0, 0)),
        ],
        out_specs=pl.BlockSpec((rows, d), lambda i: (i, 0)),
        out_shape=jax.ShapeDtypeStruct((n, d), jnp.float32),
    )(feat, w1)


def _enc_kernel(alpha_ref, sp_ref, ft_ref, x_ref, w2_ref, adjb_ref, g_ref):
    s = jax.nn.sigmoid(alpha_ref[0, 0])
    c = s * sp_ref[...] + (1.0 - s) * ft_ref[...]
    adjb_ref[...] = c.astype(jnp.bfloat16)
    h = jnp.maximum(jnp.dot(c, x_ref[...], preferred_element_type=jnp.float32), 0.0)
    g_ref[...] = jnp.dot(h, w2_ref[...], preferred_element_type=jnp.float32)


def _encode(alpha, sp, ft, x, w2):
    rows = _ROWS_ENC
    return pl.pallas_call(
        _enc_kernel,
        grid=(_N // rows,),
        in_specs=[
            pl.BlockSpec((1, 1), lambda i: (0, 0)),
            pl.BlockSpec((rows, _N), lambda i: (i, 0)),
            pl.BlockSpec((rows, _N), lambda i: (i, 0)),
            pl.BlockSpec((_N, _D), lambda i: (0, 0)),
            pl.BlockSpec((_D, _D), lambda i: (0, 0)),
        ],
        out_specs=[
            pl.BlockSpec((rows, _N), lambda i: (i, 0)),
            pl.BlockSpec((rows, _D), lambda i: (i, 0)),
        ],
        out_shape=[
            jax.ShapeDtypeStruct((_N, _N), jnp.bfloat16),
            jax.ShapeDtypeStruct((_N, _D), jnp.float32),
        ],
    )(jnp.reshape(alpha, (1, 1)), sp, ft, x, w2)


def _layernorm(x, g, b):
    mu = jnp.mean(x, axis=-1, keepdims=True)
    v = jnp.mean((x - mu) ** 2, axis=-1, keepdims=True)
    return (x - mu) * jax.lax.rsqrt(v + 1e-5) * g + b


def _dec_kernel(adjb_ref, y_ref, g_ref, b_ref, wq_ref, bq_ref, wk_ref, bk_ref,
                wv_ref, bv_ref, emb_ref, q_ref, k_ref, v_ref):
    acc = jnp.dot(adjb_ref[...], y_ref[...], preferred_element_type=jnp.float32)
    e = _layernorm(acc, g_ref[...], b_ref[...])
    emb_ref[...] = e
    q = jnp.dot(e, wq_ref[...], preferred_element_type=jnp.float32) + bq_ref[...]
    q_ref[...] = q.astype(jnp.bfloat16)
    k = jnp.dot(e, wk_ref[...], preferred_element_type=jnp.float32) + bk_ref[...]
    k_ref[...] = k.astype(jnp.bfloat16)
    v_ref[...] = jnp.dot(e, wv_ref[...], preferred_element_type=jnp.float32) + bv_ref[...]


def _decode(adjb, y, g, b, wq, bq, wk, bk, wv, bv):
    rows = _ROWS_DEC
    vec = lambda a: jnp.reshape(a, (1, _D))
    mat_spec = pl.BlockSpec((_D, _D), lambda i: (0, 0))
    vec_spec = pl.BlockSpec((1, _D), lambda i: (0, 0))
    out_spec = pl.BlockSpec((rows, _D), lambda i: (i, 0))
    out_f32 = jax.ShapeDtypeStruct((_N, _D), jnp.float32)
    out_bf16 = jax.ShapeDtypeStruct((_N, _D), jnp.bfloat16)
    return pl.pallas_call(
        _dec_kernel,
        grid=(_N // rows,),
        in_specs=[
            pl.BlockSpec((rows, _N), lambda i: (i, 0)),
            pl.BlockSpec((_N, _D), lambda i: (0, 0)),
            vec_spec, vec_spec,
            mat_spec, vec_spec, mat_spec, vec_spec, mat_spec, vec_spec,
        ],
        out_specs=[out_spec, out_spec, out_spec, out_spec],
        out_shape=[out_f32, out_bf16, out_bf16, out_f32],
    )(adjb, y.astype(jnp.bfloat16), vec(g), vec(b),
      wq, vec(bq), wk, vec(bk), wv, vec(bv))


def _attend(q_ref, k_ref, v_ref, wo_ref, bo_ref, scale):
    dn = (((1,), (1,)), ((), ()))
    s = jax.lax.dot_general(q_ref[...], k_ref[...], dn,
                            preferred_element_type=jnp.float32) * scale
    m = jnp.max(s, axis=-1, keepdims=True)
    p = jnp.exp(s - m)
    r = jnp.sum(p, axis=-1, keepdims=True)
    av = jnp.dot(p, v_ref[...], preferred_element_type=jnp.float32) / r
    return jnp.dot(av, wo_ref[...], preferred_element_type=jnp.float32) + bo_ref[...]


def _fuse_kernel(e1_ref, e2_ref, q12_ref, q21_ref, k12_ref, v12_ref, k21_ref,
                 v21_ref, wo12_ref, bo12_ref, wo21_ref, bo21_ref,
                 n1g_ref, n1b_ref, n2g_ref, n2b_ref,
                 wg_ref, bg_ref, wa_ref, ba_ref, wb_ref, bb_ref,
                 lg_ref, lb_ref, o_ref):
    scale = 1.0 / math.sqrt(float(_D))
    c1 = _attend(q12_ref, k12_ref, v12_ref, wo12_ref, bo12_ref, scale)
    c2 = _attend(q21_ref, k21_ref, v21_ref, wo21_ref, bo21_ref, scale)

    e1 = e1_ref[...]
    e2 = e2_ref[...]
    e1e = _layernorm(e1 + 0.2 * c1, n1g_ref[...], n1b_ref[...])
    e2e = _layernorm(e2 + 0.2 * c2, n2g_ref[...], n2b_ref[...])
    cat = jnp.concatenate([e1, e2, e1e, e2e], axis=-1)
    gate = jax.nn.sigmoid(
        jnp.dot(cat, wg_ref[...], preferred_element_type=jnp.float32) + bg_ref[...])
    ha = jnp.dot(cat, wa_ref[...], preferred_element_type=jnp.float32) + ba_ref[...]
    h = ha * 0.5 * (1.0 + jax.lax.erf(ha * (1.0 / math.sqrt(2.0))))
    h = jnp.dot(h, wb_ref[...], preferred_element_type=jnp.float32) + bb_ref[...]
    h = _layernorm(h, lg_ref[...], lb_ref[...])
    o_ref[...] = gate * h + (1.0 - gate) * (e1 + e2) * 0.5


def _fuse(e1, e2, q12, q21, k12, v12, k21, v21, p):
    rows = _ROWS_ATT
    vec = lambda a: jnp.reshape(a, (1, -1))
    blk = pl.BlockSpec((rows, _D), lambda i: (i, 0))
    full = pl.BlockSpec((_N, _D), lambda i: (0, 0))
    mat = lambda a, b: pl.BlockSpec((a, b), lambda i: (0, 0))
    vspec = lambda w: pl.BlockSpec((1, w), lambda i: (0, 0))
    return pl.pallas_call(
        _fuse_kernel,
        grid=(_N // rows,),
        in_specs=[
            blk, blk, blk, blk, full, full, full, full,
            mat(_D, _D), vspec(_D), mat(_D, _D), vspec(_D),
            vspec(_D), vspec(_D), vspec(_D), vspec(_D),
            mat(4 * _D, _D), vspec(_D), mat(4 * _D, 2 * _D), vspec(2 * _D),
            mat(2 * _D, _D), vspec(_D),
            vspec(_D), vspec(_D),
        ],
        out_specs=blk,
        out_shape=jax.ShapeDtypeStruct((_N, _D), jnp.float32),
    )(e1, e2, q12, q21, k12, v12, k21, v21,
      p['a12_Wo'], vec(p['a12_bo']), p['a21_Wo'], vec(p['a21_bo']),
      vec(p['n1_g']), vec(p['n1_b']), vec(p['n2_g']), vec(p['n2_b']),
      p['Wg'], vec(p['bg']), p['Wa'], vec(p['ba']), p['Wb'], vec(p['bb']),
      vec(p['lnf_g']), vec(p['lnf_b']))


def kernel(features_omics1, features_omics2, adj_spatial_omics1, adj_feature_omics1,
           adj_spatial_omics2, adj_feature_omics2, params):
    p = params
    x1 = _project(features_omics1, p['e1_W1'])
    x2 = _project(features_omics2, p['e2_W1'])
    adjb1, g1 = _encode(p['e1_alpha'], adj_spatial_omics1, adj_feature_omics1,
                        x1, p['e1_W2'])
    adjb2, g2 = _encode(p['e2_alpha'], adj_spatial_omics2, adj_feature_omics2,
                        x2, p['e2_W2'])
    emb1, q12, k21, v21 = _decode(adjb1, g1, p['e1_ln_g'], p['e1_ln_b'],
                                  p['a12_Wq'], p['a12_bq'],
                                  p['a21_Wk'], p['a21_bk'],
                                  p['a21_Wv'], p['a21_bv'])
    emb2, q21, k12, v12 = _decode(adjb2, g2, p['e2_ln_g'], p['e2_ln_b'],
                                  p['a21_Wq'], p['a21_bq'],
                                  p['a12_Wk'], p['a12_bk'],
                                  p['a12_Wv'], p['a12_bv'])
    return _fuse(emb1, emb2, q12, q21, k12, v12, k21, v21, p)


# speedup vs baseline: 1.1117x; 1.0900x over previous
import math

import jax
import jax.numpy as jnp
from jax.experimental import pallas as pl
from jax.experimental.pallas import tpu as pltpu

_N = 4096
_D = 64

_ROWS_PROJ = 256
_ROWS_ENC = 256
_ROWS_DEC = 512
_ROWS_ATT = 512


def _proj_kernel(f_ref, w_ref, o_ref):
    o_ref[...] = jnp.dot(f_ref[...], w_ref[...], preferred_element_type=jnp.float32)


def _project(feat, w1):
    n, din = feat.shape
    d = w1.shape[1]
    rows = _ROWS_PROJ
    return pl.pallas_call(
        _proj_kernel,
        grid=(n // rows,),
        in_specs=[
            pl.BlockSpec((rows, din), lambda i: (i, 0)),
            pl.BlockSpec((din, d), lambda i: (0, 0)),
        ],
        out_specs=pl.BlockSpec((rows, d), lambda i: (i, 0)),
        out_shape=jax.ShapeDtypeStruct((n, d), jnp.float32),
    )(feat, w1)


def _layernorm(x, g, b):
    mu = jnp.mean(x, axis=-1, keepdims=True)
    v = jnp.mean((x - mu) ** 2, axis=-1, keepdims=True)
    return (x - mu) * jax.lax.rsqrt(v + 1e-5) * g + b


_ENC_STEPS = _N // _ROWS_ENC


def _encdec_kernel(alpha_ref, sp_ref, ft_ref, x_ref, w2_ref, g_ref, b_ref,
                   wq_ref, bq_ref, wk_ref, bk_ref, wv_ref, bv_ref,
                   emb_ref, q_ref, k_ref, v_ref, adj_vmem, gmat_vmem):
    i = pl.program_id(0)
    rows = _ROWS_ENC

    @pl.when(i < _ENC_STEPS)
    def _enc_phase():
        s = jax.nn.sigmoid(alpha_ref[0, 0])
        c = s * sp_ref[...] + (1.0 - s) * ft_ref[...]
        adj_vmem[pl.ds(i * rows, rows), :] = c.astype(jnp.bfloat16)
        h = jnp.maximum(
            jnp.dot(c, x_ref[...], preferred_element_type=jnp.float32), 0.0)
        gmat_vmem[pl.ds(i * rows, rows), :] = jnp.dot(
            h, w2_ref[...], preferred_element_type=jnp.float32).astype(jnp.bfloat16)

    @pl.when(i >= _ENC_STEPS)
    def _dec_phase():
        a = adj_vmem[pl.ds((i - _ENC_STEPS) * rows, rows), :]
        acc = jnp.dot(a, gmat_vmem[...], preferred_element_type=jnp.float32)
        e = _layernorm(acc, g_ref[...], b_ref[...])
        emb_ref[...] = e
        q = jnp.dot(e, wq_ref[...], preferred_element_type=jnp.float32) + bq_ref[...]
        q_ref[...] = q.astype(jnp.bfloat16)
        k = jnp.dot(e, wk_ref[...], preferred_element_type=jnp.float32) + bk_ref[...]
        k_ref[...] = k.astype(jnp.bfloat16)
        v_ref[...] = jnp.dot(e, wv_ref[...], preferred_element_type=jnp.float32) + bv_ref[...]


def _encode_decode(alpha, sp, ft, x, g, b, w2, wq, bq, wk, bk, wv, bv):
    rows = _ROWS_ENC
    ns = _ENC_STEPS
    vec = lambda a: jnp.reshape(a, (1, _D))
    adj_spec = pl.BlockSpec((rows, _N), lambda i: (jnp.minimum(i, ns - 1), 0))
    mat_spec = pl.BlockSpec((_D, _D), lambda i: (0, 0))
    vec_spec = pl.BlockSpec((1, _D), lambda i: (0, 0))
    out_spec = pl.BlockSpec((rows, _D), lambda i: (jnp.maximum(i - ns, 0), 0))
    out_f32 = jax.ShapeDtypeStruct((_N, _D), jnp.float32)
    out_bf16 = jax.ShapeDtypeStruct((_N, _D), jnp.bfloat16)
    return pl.pallas_call(
        _encdec_kernel,
        grid=(2 * ns,),
        in_specs=[
            pl.BlockSpec((1, 1), lambda i: (0, 0)),
            adj_spec, adj_spec,
            pl.BlockSpec((_N, _D), lambda i: (0, 0)),
            mat_spec, vec_spec, vec_spec,
            mat_spec, vec_spec, mat_spec, vec_spec, mat_spec, vec_spec,
        ],
        out_specs=[out_spec, out_spec, out_spec, out_spec],
        out_shape=[out_f32, out_bf16, out_bf16, out_f32],
        scratch_shapes=[
            pltpu.VMEM((_N, _N), jnp.bfloat16),
            pltpu.VMEM((_N, _D), jnp.bfloat16),
        ],
    )(jnp.reshape(alpha, (1, 1)), sp, ft, x, w2, vec(g), vec(b),
      wq, vec(bq), wk, vec(bk), wv, vec(bv))


def _attend(q_ref, k_ref, v_ref, wo_ref, bo_ref, scale):
    dn = (((1,), (1,)), ((), ()))
    s = jax.lax.dot_general(q_ref[...], k_ref[...], dn,
                            preferred_element_type=jnp.float32) * scale
    m = jnp.max(s, axis=-1, keepdims=True)
    p = jnp.exp(s - m)
    r = jnp.sum(p, axis=-1, keepdims=True)
    av = jnp.dot(p, v_ref[...], preferred_element_type=jnp.float32) / r
    return jnp.dot(av, wo_ref[...], preferred_element_type=jnp.float32) + bo_ref[...]


def _fuse_kernel(e1_ref, e2_ref, q12_ref, q21_ref, k12_ref, v12_ref, k21_ref,
                 v21_ref, wo12_ref, bo12_ref, wo21_ref, bo21_ref,
                 n1g_ref, n1b_ref, n2g_ref, n2b_ref,
                 wg_ref, bg_ref, wa_ref, ba_ref, wb_ref, bb_ref,
                 lg_ref, lb_ref, o_ref):
    scale = 1.0 / math.sqrt(float(_D))
    c1 = _attend(q12_ref, k12_ref, v12_ref, wo12_ref, bo12_ref, scale)
    c2 = _attend(q21_ref, k21_ref, v21_ref, wo21_ref, bo21_ref, scale)

    e1 = e1_ref[...]
    e2 = e2_ref[...]
    e1e = _layernorm(e1 + 0.2 * c1, n1g_ref[...], n1b_ref[...])
    e2e = _layernorm(e2 + 0.2 * c2, n2g_ref[...], n2b_ref[...])
    cat = jnp.concatenate([e1, e2, e1e, e2e], axis=-1)
    gate = jax.nn.sigmoid(
        jnp.dot(cat, wg_ref[...], preferred_element_type=jnp.float32) + bg_ref[...])
    ha = jnp.dot(cat, wa_ref[...], preferred_element_type=jnp.float32) + ba_ref[...]
    h = ha * 0.5 * (1.0 + jax.lax.erf(ha * (1.0 / math.sqrt(2.0))))
    h = jnp.dot(h, wb_ref[...], preferred_element_type=jnp.float32) + bb_ref[...]
    h = _layernorm(h, lg_ref[...], lb_ref[...])
    o_ref[...] = gate * h + (1.0 - gate) * (e1 + e2) * 0.5


def _fuse(e1, e2, q12, q21, k12, v12, k21, v21, p):
    rows = _ROWS_ATT
    vec = lambda a: jnp.reshape(a, (1, -1))
    blk = pl.BlockSpec((rows, _D), lambda i: (i, 0))
    full = pl.BlockSpec((_N, _D), lambda i: (0, 0))
    mat = lambda a, b: pl.BlockSpec((a, b), lambda i: (0, 0))
    vspec = lambda w: pl.BlockSpec((1, w), lambda i: (0, 0))
    return pl.pallas_call(
        _fuse_kernel,
        grid=(_N // rows,),
        in_specs=[
            blk, blk, blk, blk, full, full, full, full,
            mat(_D, _D), vspec(_D), mat(_D, _D), vspec(_D),
            vspec(_D), vspec(_D), vspec(_D), vspec(_D),
            mat(4 * _D, _D), vspec(_D), mat(4 * _D, 2 * _D), vspec(2 * _D),
            mat(2 * _D, _D), vspec(_D),
            vspec(_D), vspec(_D),
        ],
        out_specs=blk,
        out_shape=jax.ShapeDtypeStruct((_N, _D), jnp.float32),
    )(e1, e2, q12, q21, k12, v12, k21, v21,
      p['a12_Wo'], vec(p['a12_bo']), p['a21_Wo'], vec(p['a21_bo']),
      vec(p['n1_g']), vec(p['n1_b']), vec(p['n2_g']), vec(p['n2_b']),
      p['Wg'], vec(p['bg']), p['Wa'], vec(p['ba']), p['Wb'], vec(p['bb']),
      vec(p['lnf_g']), vec(p['lnf_b']))


def kernel(features_omics1, features_omics2, adj_spatial_omics1, adj_feature_omics1,
           adj_spatial_omics2, adj_feature_omics2, params):
    p = params
    x1 = _project(features_omics1, p['e1_W1'])
    x2 = _project(features_omics2, p['e2_W1'])
    emb1, q12, k21, v21 = _encode_decode(
        p['e1_alpha'], adj_spatial_omics1, adj_feature_omics1, x1,
        p['e1_ln_g'], p['e1_ln_b'], p['e1_W2'],
        p['a12_Wq'], p['a12_bq'], p['a21_Wk'], p['a21_bk'],
        p['a21_Wv'], p['a21_bv'])
    emb2, q21, k12, v12 = _encode_decode(
        p['e2_alpha'], adj_spatial_omics2, adj_feature_omics2, x2,
        p['e2_ln_g'], p['e2_ln_b'], p['e2_W2'],
        p['a21_Wq'], p['a21_bq'], p['a12_Wk'], p['a12_bk'],
        p['a12_Wv'], p['a12_bv'])
    return _fuse(emb1, emb2, q12, q21, k12, v12, k21, v21, p)
